# baseline (device time: 41180 ns/iter reference)
import jax
import jax.numpy as jnp
from jax import lax
from jax.experimental import pallas as pl
from jax.experimental.pallas import tpu as pltpu

C = 4


def _gemm_partial(dy, W, half):
    m, k = dy.shape
    d = W.shape[0]

    def body(dy_ref, w_ref, p_ref, dyv_ref, wv_ref, wb_ref, pv_ref,
             dy_sem, w_sem, p_sem):
        my_x = lax.axis_index("x")
        row0 = my_x * half

        cp_dy = pltpu.make_async_copy(
            dy_ref.at[pl.ds(row0, half), :], dyv_ref, dy_sem)
        cp_w = pltpu.make_async_copy(w_ref, wv_ref, w_sem)
        cp_w.start()
        cp_dy.start()

        cp_w.wait()
        wb_ref[...] = wv_ref[...].astype(jnp.bfloat16).T
        cp_dy.wait()
        pv_ref[...] = lax.dot_general(
            dyv_ref[...].astype(jnp.bfloat16),
            wb_ref[...],
            dimension_numbers=(((1,), (0,)), ((), ())),
            preferred_element_type=jnp.float32,
        ).astype(jnp.bfloat16)
        cp_p = pltpu.make_async_copy(pv_ref, p_ref, p_sem)
        cp_p.start()
        cp_p.wait()

    return pl.pallas_call(
        body,
        out_shape=jax.ShapeDtypeStruct((half, d), jnp.bfloat16),
        in_specs=[pl.BlockSpec(memory_space=pl.ANY),
                  pl.BlockSpec(memory_space=pl.ANY)],
        out_specs=pl.BlockSpec(memory_space=pl.ANY),
        scratch_shapes=[
            pltpu.VMEM((half, k), jnp.float32),
            pltpu.VMEM((d, k), jnp.float32),
            pltpu.VMEM((k, d), jnp.bfloat16),
            pltpu.VMEM((half, d), jnp.bfloat16),
            pltpu.SemaphoreType.DMA,
            pltpu.SemaphoreType.DMA,
            pltpu.SemaphoreType.DMA,
        ],
        compiler_params=pltpu.CompilerParams(
            vmem_limit_bytes=100 * 1024 * 1024),
    )(dy, W)


def _all_reduce(p, m, half):
    d = p.shape[1]
    rows = half // C

    def body(p_ref, out_ref, pv_ref, ov_ref, yr_ref, xs_ref, xr_ref,
             p_sem, out_sems, ys_sems, yr_sems, xs_sems, xr_sems):
        my_x = lax.axis_index("x")
        my_y = lax.axis_index("y")
        row0 = my_x * half

        cp_p = pltpu.make_async_copy(p_ref, pv_ref, p_sem)
        cp_p.start()

        def y_rdma(c):
            return pltpu.make_async_remote_copy(
                src_ref=pv_ref.at[pl.ds(c * rows, rows), :],
                dst_ref=yr_ref.at[pl.ds(c * rows, rows), :],
                send_sem=ys_sems.at[c], recv_sem=yr_sems.at[c],
                device_id=(my_x, 1 - my_y),
                device_id_type=pl.DeviceIdType.MESH,
            )

        def x_rdma(c):
            return pltpu.make_async_remote_copy(
                src_ref=xs_ref.at[pl.ds(c * rows, rows), :],
                dst_ref=xr_ref.at[pl.ds(c * rows, rows), :],
                send_sem=xs_sems.at[c], recv_sem=xr_sems.at[c],
                device_id=(1 - my_x, my_y),
                device_id_type=pl.DeviceIdType.MESH,
            )

        def out_dma(c, vrow0, grow0):
            return pltpu.make_async_copy(
                ov_ref.at[pl.ds(vrow0, rows), :],
                out_ref.at[pl.ds(grow0, rows), :],
                out_sems.at[c],
            )

        barrier_sem = pltpu.get_barrier_semaphore()
        pl.semaphore_signal(barrier_sem, inc=1, device_id=(1 - my_x, my_y),
                            device_id_type=pl.DeviceIdType.MESH)
        pl.semaphore_signal(barrier_sem, inc=1, device_id=(my_x, 1 - my_y),
                            device_id_type=pl.DeviceIdType.MESH)
        pl.semaphore_wait(barrier_sem, 2)

        cp_p.wait()
        for c in range(C):
            y_rdma(c).start()

        for c in range(C):
            y_rdma(c).wait_recv()
            rs = pl.ds(c * rows, rows)
            s = pv_ref[rs, :].astype(jnp.float32) + yr_ref[rs, :].astype(
                jnp.float32)
            xs_ref[rs, :] = s.astype(jnp.bfloat16)
            x_rdma(c).start()
            ov_ref[rs, :] = s
            out_dma(c, c * rows, row0 + c * rows).start()

        other0 = (1 - my_x) * half
        for c in range(C):
            x_rdma(c).wait_recv()
            vr = half + c * rows
            ov_ref[pl.ds(vr, rows), :] = (
                xr_ref[pl.ds(c * rows, rows), :].astype(jnp.float32)
            )
            out_dma(C + c, vr, other0 + c * rows).start()

        for c in range(2 * C):
            out_dma(c, 0, 0).wait()
        for c in range(C):
            y_rdma(c).wait_send()
            x_rdma(c).wait_send()

    return pl.pallas_call(
        body,
        out_shape=jax.ShapeDtypeStruct((m, d), jnp.float32),
        in_specs=[pl.BlockSpec(memory_space=pl.ANY)],
        out_specs=pl.BlockSpec(memory_space=pl.ANY),
        scratch_shapes=[
            pltpu.VMEM((half, d), jnp.bfloat16),
            pltpu.VMEM((m, d), jnp.float32),
            pltpu.VMEM((half, d), jnp.bfloat16),
            pltpu.VMEM((half, d), jnp.bfloat16),
            pltpu.VMEM((half, d), jnp.bfloat16),
            pltpu.SemaphoreType.DMA,
            pltpu.SemaphoreType.DMA((2 * C,)),
            pltpu.SemaphoreType.DMA((C,)),
            pltpu.SemaphoreType.DMA((C,)),
            pltpu.SemaphoreType.DMA((C,)),
            pltpu.SemaphoreType.DMA((C,)),
        ],
        compiler_params=pltpu.CompilerParams(
            collective_id=0, vmem_limit_bytes=100 * 1024 * 1024),
    )(p)


def kernel(dy, W):
    m, k = dy.shape
    half = m // 2
    p = _gemm_partial(dy, W, half)
    return _all_reduce(p, m, half)


# device time: 33555 ns/iter; 1.2272x vs baseline; 1.2272x over previous
import jax
import jax.numpy as jnp
from jax import lax
from jax.experimental import pallas as pl
from jax.experimental.pallas import tpu as pltpu

C = 8
KC = 8


def kernel(dy, W):
    m, k = dy.shape
    d = W.shape[0]
    half = m // 2
    rows = half // C
    kch = k // KC

    def body(dy_ref, w_ref, out_ref, dyv, wv, wb, pacc, pv, yr, xs, xr,
             dsems, wsems, osems, yssems, yrsems, xssems, xrsems):
        my_x = lax.axis_index("x")
        my_y = lax.axis_index("y")
        row0 = my_x * half

        cps = []
        for kc in range(KC):
            ks = pl.ds(kc * kch, kch)
            cw = pltpu.make_async_copy(w_ref.at[:, ks], wv.at[kc],
                                       wsems.at[kc])
            cw.start()
            cd = pltpu.make_async_copy(
                dy_ref.at[pl.ds(row0, half), ks], dyv.at[:, ks],
                dsems.at[kc])
            cd.start()
            cps.append((cw, cd))

        def y_rdma(c):
            return pltpu.make_async_remote_copy(
                src_ref=pv.at[pl.ds(c * rows, rows), :],
                dst_ref=yr.at[pl.ds(c * rows, rows), :],
                send_sem=yssems.at[c], recv_sem=yrsems.at[c],
                device_id=(my_x, 1 - my_y),
                device_id_type=pl.DeviceIdType.MESH)

        def x_rdma(c):
            return pltpu.make_async_remote_copy(
                src_ref=xs.at[pl.ds(c * rows, rows), :],
                dst_ref=xr.at[pl.ds(c * rows, rows), :],
                send_sem=xssems.at[c], recv_sem=xrsems.at[c],
                device_id=(1 - my_x, my_y),
                device_id_type=pl.DeviceIdType.MESH)

        def out_dma(i, src, c, grow0):
            return pltpu.make_async_copy(
                src.at[pl.ds(c * rows, rows), :],
                out_ref.at[pl.ds(grow0, rows), :], osems.at[i])

        barrier_sem = pltpu.get_barrier_semaphore()
        pl.semaphore_signal(barrier_sem, inc=1, device_id=(1 - my_x, my_y),
                            device_id_type=pl.DeviceIdType.MESH)
        pl.semaphore_signal(barrier_sem, inc=1, device_id=(my_x, 1 - my_y),
                            device_id_type=pl.DeviceIdType.MESH)

        dn = (((1,), (0,)), ((), ()))

        for kc in range(KC - 1):
            ks = pl.ds(kc * kch, kch)
            cps[kc][0].wait()
            wb[ks, :] = wv[kc].astype(jnp.bfloat16).T
            cps[kc][1].wait()
            pk = lax.dot_general(
                dyv[:, ks].astype(jnp.bfloat16), wb[ks, :],
                dimension_numbers=dn, preferred_element_type=jnp.float32)
            pacc[...] = pk if kc == 0 else pacc[...] + pk

        lk = KC - 1
        ks = pl.ds(lk * kch, kch)
        cps[lk][0].wait()
        wb[ks, :] = wv[lk].astype(jnp.bfloat16).T
        cps[lk][1].wait()
        pl.semaphore_wait(barrier_sem, 2)
        for c in range(C):
            rs = pl.ds(c * rows, rows)
            pk = lax.dot_general(
                dyv[rs, ks].astype(jnp.bfloat16), wb[ks, :],
                dimension_numbers=dn, preferred_element_type=jnp.float32)
            pv[rs, :] = (pacc[rs, :] + pk).astype(jnp.bfloat16)
            y_rdma(c).start()

        for c in range(C):
            y_rdma(c).wait_recv()
            rs = pl.ds(c * rows, rows)
            s = pv[rs, :].astype(jnp.float32) + yr[rs, :].astype(jnp.float32)
            xs[rs, :] = s.astype(jnp.bfloat16)
            x_rdma(c).start()
            out_dma(c, xs, c, row0 + c * rows).start()

        other0 = (1 - my_x) * half
        for c in range(C):
            x_rdma(c).wait_recv()
            out_dma(C + c, xr, c, other0 + c * rows).start()

        for i in range(2 * C):
            out_dma(i, xs, 0, 0).wait()
        for c in range(C):
            y_rdma(c).wait_send()
            x_rdma(c).wait_send()

    return pl.pallas_call(
        body,
        out_shape=jax.ShapeDtypeStruct((m, d), jnp.bfloat16),
        in_specs=[pl.BlockSpec(memory_space=pl.ANY),
                  pl.BlockSpec(memory_space=pl.ANY)],
        out_specs=pl.BlockSpec(memory_space=pl.ANY),
        scratch_shapes=[
            pltpu.VMEM((half, k), jnp.float32),
            pltpu.VMEM((KC, d, k // KC), jnp.float32),
            pltpu.VMEM((k, d), jnp.bfloat16),
            pltpu.VMEM((half, d), jnp.float32),
            pltpu.VMEM((half, d), jnp.bfloat16),
            pltpu.VMEM((half, d), jnp.bfloat16),
            pltpu.VMEM((half, d), jnp.bfloat16),
            pltpu.VMEM((half, d), jnp.bfloat16),
            pltpu.SemaphoreType.DMA((KC,)),
            pltpu.SemaphoreType.DMA((KC,)),
            pltpu.SemaphoreType.DMA((2 * C,)),
            pltpu.SemaphoreType.DMA((C,)),
            pltpu.SemaphoreType.DMA((C,)),
            pltpu.SemaphoreType.DMA((C,)),
            pltpu.SemaphoreType.DMA((C,)),
        ],
        compiler_params=pltpu.CompilerParams(
            collective_id=0, vmem_limit_bytes=100 * 1024 * 1024),
    )(dy, W)
